# Initial kernel scaffold; baseline (speedup 1.0000x reference)
import jax
import jax.numpy as jnp
from jax.experimental import pallas as pl


def kernel(x, edge_index, edge_attr, W, b):
    raise NotImplementedError("write your pallas kernel here")


# speedup vs baseline: 2.9652x; 2.9652x over previous
import functools

import jax
import jax.numpy as jnp
from jax import lax
from jax.experimental import pallas as pl
from jax.experimental.pallas import tpu as pltpu
from jax.experimental.pallas import tpu_sc as plsc

N_CORES = 2
N_SUBCORES = 16
N_WORKERS = N_CORES * N_SUBCORES
CHUNK = 128
LANES = 16


def _linear(x, wt, b2):
    M, K = x.shape
    N = wt.shape[1]
    BM = 1000

    def body(x_ref, w_ref, b_ref, o_ref):
        o_ref[...] = (
            jnp.dot(x_ref[...], w_ref[...], preferred_element_type=jnp.float32)
            + b_ref[...]
        )

    return pl.pallas_call(
        body,
        grid=(M // BM,),
        in_specs=[
            pl.BlockSpec((BM, K), lambda i: (i, 0)),
            pl.BlockSpec((K, N), lambda i: (0, 0)),
            pl.BlockSpec((1, N), lambda i: (0, 0)),
        ],
        out_specs=pl.BlockSpec((BM, N), lambda i: (i, 0)),
        out_shape=jax.ShapeDtypeStruct((M, N), jnp.float32),
    )(x, wt, b2)


def _combine(p0, p1):
    M, N = p0.shape
    BM = 1000

    def body(a_ref, b_ref, o_ref):
        o_ref[...] = a_ref[...] + b_ref[...]

    return pl.pallas_call(
        body,
        grid=(M // BM,),
        in_specs=[
            pl.BlockSpec((BM, N), lambda i: (i, 0)),
            pl.BlockSpec((BM, N), lambda i: (i, 0)),
        ],
        out_specs=pl.BlockSpec((BM, N), lambda i: (i, 0)),
        out_shape=jax.ShapeDtypeStruct((M, N), jnp.float32),
    )(p0, p1)


@functools.lru_cache(maxsize=None)
def _make_spmm(n_nodes, d, chunks_per_w):
    acc_rows = 10240
    rows_per_tile = acc_rows // N_SUBCORES
    zrows = CHUNK
    mesh = plsc.VectorSubcoreMesh(core_axis_name="c", subcore_axis_name="s")

    @functools.partial(
        pl.kernel,
        mesh=mesh,
        out_type=jax.ShapeDtypeStruct((N_CORES, acc_rows, d), jnp.float32),
        scratch_types=[
            pltpu.VMEM_SHARED((acc_rows, d), jnp.float32),
            pltpu.VMEM((chunks_per_w, CHUNK), jnp.int32),
            pltpu.VMEM((chunks_per_w, CHUNK), jnp.int32),
            pltpu.VMEM((chunks_per_w, CHUNK), jnp.float32),
            pltpu.VMEM((CHUNK, d), jnp.float32),
            pltpu.SemaphoreType.DMA,
        ],
    )
    def spmm(h_hbm, src_hbm, dst_hbm, attr_hbm, out_hbm,
             acc, src_v, dst_v, attr_v, rbuf, sem):
        c = lax.axis_index("c")
        s = lax.axis_index("s")
        wid = s * N_CORES + c
        base = wid * chunks_per_w

        pltpu.sync_copy(src_hbm.at[pl.ds(base, chunks_per_w)], src_v)
        pltpu.sync_copy(dst_hbm.at[pl.ds(base, chunks_per_w)], dst_v)
        pltpu.sync_copy(attr_hbm.at[pl.ds(base, chunks_per_w)], attr_v)

        zv = jnp.zeros((LANES,), jnp.float32)

        def zrow(i, carry):
            for q in range(d // LANES):
                rbuf[i, pl.ds(q * LANES, LANES)] = zv
            return carry

        lax.fori_loop(0, zrows, zrow, 0)

        def zacc(k, carry):
            pltpu.sync_copy(
                rbuf, acc.at[pl.ds(s * rows_per_tile + k * zrows, zrows)]
            )
            return carry

        lax.fori_loop(0, rows_per_tile // zrows, zacc, 0)
        plsc.subcore_barrier()

        def chunk_body(j, carry):
            pltpu.async_copy(h_hbm.at[src_v.at[j]], rbuf, sem).wait()

            def group_body(g, carry2):
                av = attr_v[j, pl.ds(g * LANES, LANES)]
                for i in range(LANES):
                    a = av[i]
                    e = g * LANES + i
                    for q in range(d // LANES):
                        rbuf[e, pl.ds(q * LANES, LANES)] = (
                            rbuf[e, pl.ds(q * LANES, LANES)] * a
                        )
                return carry2

            lax.fori_loop(0, CHUNK // LANES, group_body, 0)
            pltpu.sync_copy(rbuf, acc.at[dst_v.at[j]], add=True)
            return carry

        lax.fori_loop(0, chunks_per_w, chunk_body, 0)
        plsc.subcore_barrier()

        pltpu.sync_copy(
            acc.at[pl.ds(s * rows_per_tile, rows_per_tile)],
            out_hbm.at[c, pl.ds(s * rows_per_tile, rows_per_tile)],
        )

    return spmm


def kernel(x, edge_index, edge_attr, W, b):
    n_nodes, d = x.shape
    n_edges = edge_attr.shape[0]
    chunks_per_w = -(-n_edges // (N_WORKERS * CHUNK))
    chunks_per_w = -(-chunks_per_w // 8) * 8
    e_pad = N_WORKERS * chunks_per_w * CHUNK

    dst = jnp.pad(edge_index[0], (0, e_pad - n_edges)).reshape(-1, CHUNK)
    src = jnp.pad(edge_index[1], (0, e_pad - n_edges)).reshape(-1, CHUNK)
    attr = jnp.pad(edge_attr, (0, e_pad - n_edges)).reshape(-1, CHUNK)

    h = _linear(x, W.T, b.reshape(1, -1))
    spmm = _make_spmm(n_nodes, d, chunks_per_w)
    for _ in range(3):
        partials = spmm(h, src, dst, attr)
        h = _combine(partials[0, :n_nodes], partials[1, :n_nodes])
    return h


# speedup vs baseline: 3.4233x; 1.1545x over previous
import functools

import jax
import jax.numpy as jnp
from jax import lax
from jax.experimental import pallas as pl
from jax.experimental.pallas import tpu as pltpu
from jax.experimental.pallas import tpu_sc as plsc

N_CORES = 2
N_SUBCORES = 16
N_WORKERS = N_CORES * N_SUBCORES
CHUNK = 128
LANES = 16


def _linear(x, wt, b2):
    M, K = x.shape
    N = wt.shape[1]
    BM = 1000

    def body(x_ref, w_ref, b_ref, o_ref):
        o_ref[...] = (
            jnp.dot(x_ref[...], w_ref[...], preferred_element_type=jnp.float32)
            + b_ref[...]
        )

    return pl.pallas_call(
        body,
        grid=(M // BM,),
        in_specs=[
            pl.BlockSpec((BM, K), lambda i: (i, 0)),
            pl.BlockSpec((K, N), lambda i: (0, 0)),
            pl.BlockSpec((1, N), lambda i: (0, 0)),
        ],
        out_specs=pl.BlockSpec((BM, N), lambda i: (i, 0)),
        out_shape=jax.ShapeDtypeStruct((M, N), jnp.float32),
    )(x, wt, b2)


def _combine(p0, p1):
    M, N = p0.shape
    BM = 1000

    def body(a_ref, b_ref, o_ref):
        o_ref[...] = a_ref[...] + b_ref[...]

    return pl.pallas_call(
        body,
        grid=(M // BM,),
        in_specs=[
            pl.BlockSpec((BM, N), lambda i: (i, 0)),
            pl.BlockSpec((BM, N), lambda i: (i, 0)),
        ],
        out_specs=pl.BlockSpec((BM, N), lambda i: (i, 0)),
        out_shape=jax.ShapeDtypeStruct((M, N), jnp.float32),
    )(p0, p1)


@functools.lru_cache(maxsize=None)
def _make_spmm(n_nodes, d, c0pw, c1pw):
    acc_rows = 10240
    rows_per_tile = acc_rows // N_SUBCORES
    zrows = CHUNK
    wch = 8
    mesh = plsc.VectorSubcoreMesh(core_axis_name="c", subcore_axis_name="s")

    @functools.partial(
        pl.kernel,
        mesh=mesh,
        out_type=jax.ShapeDtypeStruct((N_CORES, acc_rows, d), jnp.float32),
        scratch_types=[
            pltpu.VMEM_SHARED((acc_rows, d), jnp.float32),
            pltpu.VMEM((wch, CHUNK), jnp.int32),
            pltpu.VMEM((wch, CHUNK), jnp.int32),
            pltpu.VMEM((wch, CHUNK), jnp.float32),
            pltpu.VMEM((CHUNK, d), jnp.float32),
            pltpu.SemaphoreType.DMA,
        ],
    )
    def spmm(h_hbm, src_hbm, dst_hbm, attr_hbm, out_hbm,
             acc, src_v, dst_v, attr_v, rbuf, sem):
        c = lax.axis_index("c")
        s = lax.axis_index("s")
        base = jnp.where(c == 0, s * c0pw, N_SUBCORES * c0pw + s * c1pw)

        zv = jnp.zeros((LANES,), jnp.float32)

        def zrow(i, carry):
            for q in range(d // LANES):
                rbuf[i, pl.ds(q * LANES, LANES)] = zv
            return carry

        lax.fori_loop(0, zrows, zrow, 0)

        def zacc(k, carry):
            pltpu.sync_copy(
                rbuf, acc.at[pl.ds(s * rows_per_tile + k * zrows, zrows)]
            )
            return carry

        lax.fori_loop(0, rows_per_tile // zrows, zacc, 0)
        plsc.subcore_barrier()

        def win_body(w, carry):
            wb = base + w * wch
            pltpu.sync_copy(src_hbm.at[pl.ds(wb, wch)], src_v)
            pltpu.sync_copy(dst_hbm.at[pl.ds(wb, wch)], dst_v)
            pltpu.sync_copy(attr_hbm.at[pl.ds(wb, wch)], attr_v)

            def chunk_body(j, carry1):
                pltpu.async_copy(h_hbm.at[src_v.at[j]], rbuf, sem).wait()

                def group_body(g, carry2):
                    av = attr_v[j, pl.ds(g * LANES, LANES)]
                    for i in range(LANES):
                        a = av[i]
                        e = g * LANES + i
                        for q in range(d // LANES):
                            rbuf[e, pl.ds(q * LANES, LANES)] = (
                                rbuf[e, pl.ds(q * LANES, LANES)] * a
                            )
                    return carry2

                lax.fori_loop(0, CHUNK // LANES, group_body, 0)
                pltpu.sync_copy(rbuf, acc.at[dst_v.at[j]], add=True)
                return carry1

            lax.fori_loop(0, wch, chunk_body, 0)
            return carry

        @pl.when(c == 0)
        def _():
            lax.fori_loop(0, c0pw // wch, win_body, 0)

        @pl.when(c == 1)
        def _():
            lax.fori_loop(0, c1pw // wch, win_body, 0)

        plsc.subcore_barrier()

        pltpu.sync_copy(
            acc.at[pl.ds(s * rows_per_tile, rows_per_tile)],
            out_hbm.at[c, pl.ds(s * rows_per_tile, rows_per_tile)],
        )

    return spmm


def kernel(x, edge_index, edge_attr, W, b):
    n_nodes, d = x.shape
    n_edges = edge_attr.shape[0]
    total_chunks = -(-n_edges // (N_WORKERS * CHUNK)) * N_WORKERS
    total_chunks = -(-total_chunks // (16 * 8)) * (16 * 8)
    per_sc = total_chunks // N_SUBCORES
    c0pw = min((per_sc * 7 // 10 // 8) * 8, per_sc - 8)
    c1pw = per_sc - c0pw
    e_pad = total_chunks * CHUNK

    dst = jnp.pad(edge_index[0], (0, e_pad - n_edges)).reshape(-1, CHUNK)
    src = jnp.pad(edge_index[1], (0, e_pad - n_edges)).reshape(-1, CHUNK)
    attr = jnp.pad(edge_attr, (0, e_pad - n_edges)).reshape(-1, CHUNK)

    h = _linear(x, W.T, b.reshape(1, -1))
    spmm = _make_spmm(n_nodes, d, c0pw, c1pw)
    for _ in range(3):
        partials = spmm(h, src, dst, attr)
        h = _combine(partials[0, :n_nodes], partials[1, :n_nodes])
    return h
